# baseline (speedup 1.0000x reference)
import jax
import jax.numpy as jnp
from jax.experimental import pallas as pl

_SPLIT = 2


def _heads_body(x0, w0, b0, x1, w1, b1, x2, w2, b2, o0, o1, o2):
    dn = (((1,), (1,)), ((), ()))
    o0[0] = jax.lax.dot_general(
        x0[0], w0[...], dn, preferred_element_type=jnp.float32) + b0[...]
    o1[0] = jax.lax.dot_general(
        x1[0], w1[...], dn, preferred_element_type=jnp.float32) + b1[...]
    o2[0] = jax.lax.dot_general(
        x2[0], w2[...], dn, preferred_element_type=jnp.float32) + b2[...]


def kernel(feats_0, feats_1, feats_2, W0, b0, W1, b1, W2, b2):
    B = feats_0.shape[0]
    shapes = [feats_0.shape, feats_1.shape, feats_2.shape]
    xs = [jnp.transpose(f, (0, 2, 3, 1)).reshape(
              f.shape[0], f.shape[2] * f.shape[3], f.shape[1])
          for f in (feats_0, feats_1, feats_2)]
    ws = [W0, W1, W2]
    bs = [b.reshape(1, -1) for b in (b0, b1, b2)]
    out_dim = W0.shape[0]

    def feat_spec(x):
        return pl.BlockSpec((1, x.shape[1] // _SPLIT, x.shape[2]),
                            lambda b, j: (b, j, 0))

    def full_spec(a):
        return pl.BlockSpec(a.shape, lambda b, j: (0,) * a.ndim)

    in_specs = []
    operands = []
    for x, w, bia in zip(xs, ws, bs):
        operands.extend([x, w, bia])
        in_specs.extend([feat_spec(x), full_spec(w), full_spec(bia)])

    out_shapes = [jax.ShapeDtypeStruct((B, x.shape[1], out_dim), jnp.float32)
                  for x in xs]
    out_specs = [pl.BlockSpec((1, x.shape[1] // _SPLIT, out_dim),
                              lambda b, j: (b, j, 0))
                 for x in xs]

    outs = pl.pallas_call(
        _heads_body,
        grid=(B, _SPLIT),
        in_specs=in_specs,
        out_specs=out_specs,
        out_shape=out_shapes,
    )(*operands)

    return tuple(
        jnp.transpose(o.reshape(s[0], s[2], s[3], out_dim), (0, 3, 1, 2))
        for o, s in zip(outs, shapes)
    )


# speedup vs baseline: 1.1150x; 1.1150x over previous
import jax
import jax.numpy as jnp
from jax.experimental import pallas as pl


_ROWS = 2


def _heads_body(x0, w0, b0, x1, w1, b1, x2, w2, b2, o0, o1, o2):
    dn = (((1,), (1,)), ((), ()))
    for r in range(_ROWS):
        o0[r] = jax.lax.dot_general(
            x0[r], w0[...], dn, preferred_element_type=jnp.float32) + b0[...]
        o1[r] = jax.lax.dot_general(
            x1[r], w1[...], dn, preferred_element_type=jnp.float32) + b1[...]
        o2[r] = jax.lax.dot_general(
            x2[r], w2[...], dn, preferred_element_type=jnp.float32) + b2[...]


def kernel(feats_0, feats_1, feats_2, W0, b0, W1, b1, W2, b2):
    B = feats_0.shape[0]
    shapes = [feats_0.shape, feats_1.shape, feats_2.shape]
    xs = [jnp.transpose(f, (0, 2, 3, 1)).reshape(
              f.shape[0], f.shape[2] * f.shape[3], f.shape[1])
          for f in (feats_0, feats_1, feats_2)]
    ws = [W0, W1, W2]
    bs = [b.reshape(1, -1) for b in (b0, b1, b2)]
    out_dim = W0.shape[0]

    def feat_spec(x):
        return pl.BlockSpec((_ROWS, x.shape[1], x.shape[2]),
                            lambda b: (b, 0, 0))

    def full_spec(a):
        return pl.BlockSpec(a.shape, lambda b: (0,) * a.ndim)

    in_specs = []
    operands = []
    for x, w, bia in zip(xs, ws, bs):
        operands.extend([x, w, bia])
        in_specs.extend([feat_spec(x), full_spec(w), full_spec(bia)])

    out_shapes = [jax.ShapeDtypeStruct((B, x.shape[1], out_dim), jnp.float32)
                  for x in xs]
    out_specs = [pl.BlockSpec((_ROWS, x.shape[1], out_dim),
                              lambda b: (b, 0, 0))
                 for x in xs]

    outs = pl.pallas_call(
        _heads_body,
        grid=(B // _ROWS,),
        in_specs=in_specs,
        out_specs=out_specs,
        out_shape=out_shapes,
    )(*operands)

    return tuple(
        jnp.transpose(o.reshape(s[0], s[2], s[3], out_dim), (0, 3, 1, 2))
        for o, s in zip(outs, shapes)
    )


# speedup vs baseline: 1.1615x; 1.0417x over previous
import jax
import jax.numpy as jnp
from jax.experimental import pallas as pl


_ROWS = 2


def _heads_body(x0a, x0b, w0, b0, x1, w1, b1, x2, w2, b2, o0, o1, o2):
    dn = (((1,), (1,)), ((), ()))
    half = o0.shape[1] // 2
    for r in range(_ROWS):
        o0[r, :half] = jax.lax.dot_general(
            x0a[r, 0], w0[...], dn,
            preferred_element_type=jnp.float32) + b0[...]
        o0[r, half:] = jax.lax.dot_general(
            x0b[r, 0], w0[...], dn,
            preferred_element_type=jnp.float32) + b0[...]
        o1[r] = jax.lax.dot_general(
            x1[r], w1[...], dn, preferred_element_type=jnp.float32) + b1[...]
        o2[r] = jax.lax.dot_general(
            x2[r], w2[...], dn, preferred_element_type=jnp.float32) + b2[...]


def kernel(feats_0, feats_1, feats_2, W0, b0, W1, b1, W2, b2):
    B = feats_0.shape[0]
    shapes = [feats_0.shape, feats_1.shape, feats_2.shape]
    xs = [jnp.transpose(f, (0, 2, 3, 1)).reshape(
              f.shape[0], f.shape[2] * f.shape[3], f.shape[1])
          for f in (feats_0, feats_1, feats_2)]
    ws = [W0, W1, W2]
    bs = [b.reshape(1, -1) for b in (b0, b1, b2)]
    out_dim = W0.shape[0]

    def feat_spec(x):
        return pl.BlockSpec((_ROWS, x.shape[1], x.shape[2]),
                            lambda b: (b, 0, 0))

    def full_spec(a):
        return pl.BlockSpec(a.shape, lambda b: (0,) * a.ndim)

    x0v = xs[0].reshape(B, 2, xs[0].shape[1] // 2, xs[0].shape[2])
    half_spec = lambda j: pl.BlockSpec(
        (_ROWS, 1, x0v.shape[2], x0v.shape[3]), lambda b, jj=j: (b, jj, 0, 0))

    in_specs = [half_spec(0), half_spec(1),
                full_spec(ws[0]), full_spec(bs[0])]
    operands = [x0v, x0v, ws[0], bs[0]]
    for x, w, bia in zip(xs[1:], ws[1:], bs[1:]):
        operands.extend([x, w, bia])
        in_specs.extend([feat_spec(x), full_spec(w), full_spec(bia)])

    out_shapes = [jax.ShapeDtypeStruct((B, x.shape[1], out_dim), jnp.float32)
                  for x in xs]
    out_specs = [pl.BlockSpec((_ROWS, x.shape[1], out_dim),
                              lambda b: (b, 0, 0))
                 for x in xs]

    outs = pl.pallas_call(
        _heads_body,
        grid=(B // _ROWS,),
        in_specs=in_specs,
        out_specs=out_specs,
        out_shape=out_shapes,
    )(*operands)

    return tuple(
        jnp.transpose(o.reshape(s[0], s[2], s[3], out_dim), (0, 3, 1, 2))
        for o, s in zip(outs, shapes)
    )
